# baseline (speedup 1.0000x reference)
import functools
import jax
import jax.numpy as jnp
from jax import lax
from jax.experimental import pallas as pl
from jax.experimental.pallas import tpu as pltpu
from jax.experimental.pallas import tpu_sc as plsc

_N = 10000
_NP = 10240
_E = 320000
_NC, _NS = 2, 16
_NW = _NC * _NS
_CHUNK = 128
_NCHUNK = 80
_EPW = _CHUNK * _NCHUNK
_EP = _EPW * _NW


def _make_scatter_sum(D):
  mesh = plsc.VectorSubcoreMesh(core_axis_name="c", subcore_axis_name="s")
  rows_per_tile = _NP // _NS

  @functools.partial(
      pl.kernel,
      out_type=jax.ShapeDtypeStruct((_NC, _NP, D), jnp.float32),
      mesh=mesh,
      scratch_types=[
          pltpu.VMEM((_NCHUNK, _CHUNK), jnp.int32),
          pltpu.VMEM((_NCHUNK, _CHUNK), jnp.int32),
          pltpu.VMEM((_CHUNK, D), jnp.float32),
          pltpu.VMEM_SHARED((_NP, D), jnp.float32),
          pltpu.SemaphoreType.DMA,
      ],
  )
  def scatter_sum(u_hbm, src_hbm, dst_hbm, zero_hbm, out_hbm,
                  src_v, dst_v, rows_v, acc_sh, sem):
    c = lax.axis_index("c")
    s = lax.axis_index("s")
    wid = s * _NC + c

    @pl.when(s == 0)
    def _init():
      pltpu.sync_copy(zero_hbm, acc_sh)

    pltpu.sync_copy(src_hbm.at[wid], src_v)
    pltpu.sync_copy(dst_hbm.at[wid], dst_v)
    plsc.subcore_barrier()

    def body(k, carry):
      pltpu.async_copy(u_hbm.at[src_v.at[k]], rows_v, sem).wait()
      pltpu.sync_copy(rows_v, acc_sh.at[dst_v.at[k]], add=True)
      return carry

    lax.fori_loop(0, _NCHUNK, body, 0)
    plsc.subcore_barrier()

    base = s * rows_per_tile
    pltpu.sync_copy(acc_sh.at[pl.ds(base, rows_per_tile)],
                    out_hbm.at[c, pl.ds(base, rows_per_tile)])

  return scatter_sum


_scatter_sum_128 = _make_scatter_sum(128)


def kernel(x, edge_index, W_enc, b_enc, W_out, b_out, h, c_re, c_im, Wc, bc):
  n = x.shape[0]

  def _pad(v, fill):
    return jnp.concatenate(
        [v.astype(jnp.int32),
         jnp.full((_EP - _E,), fill, jnp.int32)]).reshape(_NW, _NCHUNK, _CHUNK)

  src_g, src_s = _pad(edge_index[0], 0), _pad(edge_index[0], n)
  dst_g, dst_s = _pad(edge_index[1], 0), _pad(edge_index[1], n)

  z128 = jnp.zeros((_NP, 128), jnp.float32)
  ones128 = jnp.ones((n, 128), jnp.float32)
  p_in = _scatter_sum_128(ones128, src_g, dst_s, z128)
  p_out = _scatter_sum_128(ones128, dst_g, src_s, z128)
  deg_in = jnp.maximum(p_in[0, :n, 0] + p_in[1, :n, 0], 1.0)
  deg_out = jnp.maximum(p_out[0, :n, 0] + p_out[1, :n, 0], 1.0)
  a = lax.rsqrt(deg_out)
  b = lax.rsqrt(deg_in)

  def agg(v):
    p = _scatter_sum_128(a[:, None] * v, src_g, dst_s, z128)
    return b[:, None] * (p[0, :n] + p[1, :n])

  def lap(v):
    return v - agg(v)

  x = x @ W_enc + b_enc
  K = c_re.shape[1] - 1
  for l in range(c_re.shape[0]):
    hl = h[l]
    h2 = hl * hl + 1.0
    out = c_re[l, 0] * x
    yr, yi = x, jnp.zeros_like(x)
    for j in range(1, K + 1):
      br = hl * lap(yr) - yi
      bi = hl * lap(yi) + yr
      yjr = (br * hl - bi) / h2
      yji = (bi * hl + br) / h2
      for _ in range(5):
        zr = br + hl * agg(yjr)
        zi = bi + hl * agg(yji)
        yjr = (zr * hl - zi) / h2
        yji = (zi * hl + zr) / h2
      out = out + 2.0 * (c_re[l, j] * yjr - c_im[l, j] * yji)
      yr, yi = yjr, yji
    x = out @ Wc[l] + bc[l]
  return x @ W_out + b_out


# speedup vs baseline: 2.3157x; 2.3157x over previous
import functools
import jax
import jax.numpy as jnp
from jax import lax
from jax.experimental import pallas as pl
from jax.experimental.pallas import tpu as pltpu
from jax.experimental.pallas import tpu_sc as plsc

_N = 10000
_NP = 10240
_E = 320000
_NC, _NS = 2, 16
_NW = _NC * _NS
_CHUNK = 80
_NCHUNK = 125
_EPW = _CHUNK * _NCHUNK
_EP = _EPW * _NW


def _make_scatter_sum(D):
  mesh = plsc.VectorSubcoreMesh(core_axis_name="c", subcore_axis_name="s")
  rows_per_tile = _NP // _NS

  @functools.partial(
      pl.kernel,
      out_type=jax.ShapeDtypeStruct((_NC, _NP, D), jnp.float32),
      mesh=mesh,
      scratch_types=[
          pltpu.VMEM((_NCHUNK, _CHUNK), jnp.int32),
          pltpu.VMEM((_NCHUNK, _CHUNK), jnp.int32),
          pltpu.VMEM((_CHUNK, D), jnp.float32),
          pltpu.VMEM_SHARED((_NP, D), jnp.float32),
          pltpu.SemaphoreType.DMA,
      ],
  )
  def scatter_sum(u_hbm, src_hbm, dst_hbm, zero_hbm, out_hbm,
                  src_v, dst_v, rows_v, acc_sh, sem):
    c = lax.axis_index("c")
    s = lax.axis_index("s")
    wid = s * _NC + c

    @pl.when(s == 0)
    def _init():
      pltpu.sync_copy(zero_hbm, acc_sh)

    pltpu.sync_copy(src_hbm.at[wid], src_v)
    pltpu.sync_copy(dst_hbm.at[wid], dst_v)
    plsc.subcore_barrier()

    def body(k, carry):
      pltpu.async_copy(u_hbm.at[src_v.at[k]], rows_v, sem).wait()
      pltpu.sync_copy(rows_v, acc_sh.at[dst_v.at[k]], add=True)
      return carry

    lax.fori_loop(0, _NCHUNK, body, 0)
    plsc.subcore_barrier()

    base = s * rows_per_tile
    pltpu.sync_copy(acc_sh.at[pl.ds(base, rows_per_tile)],
                    out_hbm.at[c, pl.ds(base, rows_per_tile)])

  return scatter_sum


_scatter_sum_128 = _make_scatter_sum(128)


def kernel(x, edge_index, W_enc, b_enc, W_out, b_out, h, c_re, c_im, Wc, bc):
  n = x.shape[0]

  def _pad(v, fill):
    return jnp.concatenate(
        [v.astype(jnp.int32),
         jnp.full((_EP - _E,), fill, jnp.int32)]).reshape(_NW, _NCHUNK, _CHUNK)

  src_g, src_s = _pad(edge_index[0], 0), _pad(edge_index[0], n)
  dst_g, dst_s = _pad(edge_index[1], 0), _pad(edge_index[1], n)

  z128 = jnp.zeros((_NP, 128), jnp.float32)
  ones128 = jnp.ones((n, 128), jnp.float32)
  p_in = _scatter_sum_128(ones128, src_g, dst_s, z128)
  p_out = _scatter_sum_128(ones128, dst_g, src_s, z128)
  deg_in = jnp.maximum(p_in[0, :n, 0] + p_in[1, :n, 0], 1.0)
  deg_out = jnp.maximum(p_out[0, :n, 0] + p_out[1, :n, 0], 1.0)
  a = lax.rsqrt(deg_out)
  b = lax.rsqrt(deg_in)

  def agg(v):
    p = _scatter_sum_128(a[:, None] * v, src_g, dst_s, z128)
    return b[:, None] * (p[0, :n] + p[1, :n])

  def lap(v):
    return v - agg(v)

  x = x @ W_enc + b_enc
  K = c_re.shape[1] - 1
  for l in range(c_re.shape[0]):
    hl = h[l]
    h2 = hl * hl + 1.0
    out = c_re[l, 0] * x
    yr, yi = x, jnp.zeros_like(x)
    for j in range(1, K + 1):
      br = hl * lap(yr) - yi
      bi = hl * lap(yi) + yr
      yjr = (br * hl - bi) / h2
      yji = (bi * hl + br) / h2
      for _ in range(5):
        zr = br + hl * agg(yjr)
        zi = bi + hl * agg(yji)
        yjr = (zr * hl - zi) / h2
        yji = (zi * hl + zr) / h2
      out = out + 2.0 * (c_re[l, j] * yjr - c_im[l, j] * yji)
      yr, yi = yjr, yji
    x = out @ Wc[l] + bc[l]
  return x @ W_out + b_out
